# baseline (speedup 1.0000x reference)
import functools

import jax
import jax.numpy as jnp
import numpy as np
from jax import lax
from jax.experimental import pallas as pl
from jax.experimental.pallas import tpu as pltpu
from jax.experimental.pallas import tpu_sc as plsc

_B = 64
_SPATIAL = 1024
_CDIM = 768
_SS = 32
_PS = 16
_NPOOL = 256
_TDIM = 32
_NC = 2
_NS = 16
_NW = _NC * _NS
_B_PER_W = _B // _NW
_BLK_S = 64
_NBLK = _SPATIAL // _BLK_S

_CH = np.linspace(0, _CDIM - 1, _TDIM).astype(np.int32)


def _issue_block(feat_hbm, b, r_grp, buf, sem):
    pltpu.async_copy(
        feat_hbm.at[pl.ds(b * _SPATIAL + r_grp * _BLK_S, _BLK_S), :],
        buf,
        sem,
    )


def _drain_block(feat_hbm, buf, sem):
    pltpu.make_async_copy(
        feat_hbm.at[pl.ds(0, _BLK_S), :],
        buf,
        sem,
    ).wait()


def _compute_block(ch_v, buf, out_v):
    for h in range(2):
        chv = ch_v[h]
        for c_col in range(_PS):
            acc = None
            for r in (2 * c_col, 2 * c_col + 1,
                      _SS + 2 * c_col, _SS + 2 * c_col + 1):
                rv = jnp.full((16,), r, jnp.int32)
                v = plsc.load_gather(buf, [rv, chv])
                acc = v if acc is None else acc + v
            out_v[c_col, pl.ds(h * 16, 16)] = acc * jnp.float32(0.25)


def _sc_body(feat_hbm, ch_hbm, out_hbm, ch_v, buf_a, buf_b, out_v,
             sem_a, sem_b):
    wid = lax.axis_index("s") * _NC + lax.axis_index("c")
    pltpu.sync_copy(ch_hbm, ch_v)

    for b_local in range(_B_PER_W):
        b = wid * _B_PER_W + b_local
        _issue_block(feat_hbm, b, jnp.int32(0), buf_a, sem_a)

        def blk_body(g, _):
            r_even = g * 2
            _issue_block(feat_hbm, b, r_even + 1, buf_b, sem_b)
            _drain_block(feat_hbm, buf_a, sem_a)
            _compute_block(ch_v, buf_a, out_v)
            pltpu.sync_copy(out_v, out_hbm.at[b, pl.ds(r_even * _PS, _PS)])

            @pl.when(g < _NBLK // 2 - 1)
            def _prefetch():
                _issue_block(feat_hbm, b, r_even + 2, buf_a, sem_a)

            _drain_block(feat_hbm, buf_b, sem_b)
            _compute_block(ch_v, buf_b, out_v)
            pltpu.sync_copy(out_v, out_hbm.at[b, pl.ds((r_even + 1) * _PS, _PS)])
            return _

        lax.fori_loop(0, _NBLK // 2, blk_body, None)


def kernel(features):
    b, spatial, c = features.shape
    feat2d = features.reshape(b * spatial, c)
    chans = jnp.asarray(_CH.reshape(2, 16))

    sc_call = functools.partial(
        pl.kernel,
        mesh=plsc.VectorSubcoreMesh(core_axis_name="c", subcore_axis_name="s"),
        compiler_params=pltpu.CompilerParams(
            use_tc_tiling_on_sc=False, needs_layout_passes=False),
        out_type=jax.ShapeDtypeStruct((_B, _NPOOL, _TDIM), jnp.float32),
        scratch_types=[
            pltpu.VMEM((2, 16), jnp.int32),
            pltpu.VMEM((_BLK_S, _CDIM), jnp.float32),
            pltpu.VMEM((_BLK_S, _CDIM), jnp.float32),
            pltpu.VMEM((_PS, _TDIM), jnp.float32),
            pltpu.SemaphoreType.DMA,
            pltpu.SemaphoreType.DMA,
        ],
    )(_sc_body)
    return sc_call(feat2d, chans)


# speedup vs baseline: 2.7090x; 2.7090x over previous
import functools

import jax
import jax.numpy as jnp
import numpy as np
from jax import lax
from jax.experimental import pallas as pl
from jax.experimental.pallas import tpu as pltpu
from jax.experimental.pallas import tpu_sc as plsc

_B = 64
_SPATIAL = 1024
_CDIM = 768
_SS = 32
_PS = 16
_NPOOL = 256
_TDIM = 32
_NC = 2
_NS = 16
_NW = _NC * _NS
_B_PER_W = _B // _NW
_BLK_S = 64
_NBLK = _SPATIAL // _BLK_S

_CH = np.linspace(0, _CDIM - 1, _TDIM).astype(np.int32)


def _issue_block(feat_hbm, b, r_grp, buf, sem):
    pltpu.async_copy(
        feat_hbm.at[pl.ds(b * _SPATIAL + r_grp * _BLK_S, _BLK_S), :],
        buf,
        sem,
    )


def _drain_block(feat_hbm, buf, sem):
    pltpu.make_async_copy(
        feat_hbm.at[pl.ds(0, _BLK_S), :],
        buf,
        sem,
    ).wait()


def _compute_block(ch_v, buf, out_v):
    for h in range(2):
        chv = ch_v[h]
        for c_col in range(_PS):
            acc = None
            for r in (2 * c_col, 2 * c_col + 1,
                      _SS + 2 * c_col, _SS + 2 * c_col + 1):
                rv = jnp.full((16,), r, jnp.int32)
                v = plsc.load_gather(buf, [rv, chv])
                acc = v if acc is None else acc + v
            out_v[c_col, pl.ds(h * 16, 16)] = acc * jnp.float32(0.25)


def _sc_body(feat_hbm, ch_hbm, out_hbm, ch_v, buf_a, buf_b, out_v,
             sem_a, sem_b):
    wid = lax.axis_index("s") * _NC + lax.axis_index("c")
    pltpu.sync_copy(ch_hbm, ch_v)

    for b_local in range(_B_PER_W):
        b = wid * _B_PER_W + b_local
        _issue_block(feat_hbm, b, jnp.int32(0), buf_a, sem_a)

        def blk_body(g, _):
            r_even = g * 2
            _issue_block(feat_hbm, b, r_even + 1, buf_b, sem_b)
            _drain_block(feat_hbm, buf_a, sem_a)
            _compute_block(ch_v, buf_a, out_v)
            pltpu.sync_copy(out_v, out_hbm.at[b, pl.ds(r_even * _PS, _PS)])

            @pl.when(g < _NBLK // 2 - 1)
            def _prefetch():
                _issue_block(feat_hbm, b, r_even + 2, buf_a, sem_a)

            _drain_block(feat_hbm, buf_b, sem_b)
            _compute_block(ch_v, buf_b, out_v)
            pltpu.sync_copy(out_v, out_hbm.at[b, pl.ds((r_even + 1) * _PS, _PS)])
            return _

        lax.fori_loop(0, _NBLK // 2, blk_body, None)


def kernel(features):
    b, spatial, c = features.shape
    feat2d = features.reshape(b * spatial, c)
    chans = jnp.asarray(_CH.reshape(2, 16))

    sc_call = functools.partial(
        pl.kernel,
        mesh=plsc.VectorSubcoreMesh(core_axis_name="c", subcore_axis_name="s"),
        compiler_params=pltpu.CompilerParams(
            use_tc_tiling_on_sc=True, needs_layout_passes=False),
        out_type=jax.ShapeDtypeStruct((_B, _NPOOL, _TDIM), jnp.float32),
        scratch_types=[
            pltpu.VMEM((2, 16), jnp.int32),
            pltpu.VMEM((_BLK_S, _CDIM), jnp.float32),
            pltpu.VMEM((_BLK_S, _CDIM), jnp.float32),
            pltpu.VMEM((_PS, _TDIM), jnp.float32),
            pltpu.SemaphoreType.DMA,
            pltpu.SemaphoreType.DMA,
        ],
    )(_sc_body)
    return sc_call(feat2d, chans)
